# baseline (speedup 1.0000x reference)
import jax
import jax.numpy as jnp
from jax import lax
from jax.experimental import pallas as pl
from jax.experimental.pallas import tpu as pltpu
from jax.experimental.pallas import tpu_sc as plsc

N = 10000
D = 128
E = 320000
NC = 2
NS = 16
NW = NC * NS
AR = 10112
EPW = E // NW
K = 176
CHP = 58
EPTP = CHP * K
ZR = AR // NS
CPT = 624
CREM = N - CPT * NS


def _make_sc_aggregate(compute_cnt, K=K, CHP=CHP, AR=AR):
    ZR = AR // NS
    mesh = plsc.VectorSubcoreMesh(core_axis_name="c", subcore_axis_name="s",
                                  num_cores=NC, num_subcores=NS)
    out_type = [jax.ShapeDtypeStruct((NC, N, D), jnp.float32)]
    scratch = [
        pltpu.VMEM((K,), jnp.int32),
        pltpu.VMEM((K,), jnp.int32),
        pltpu.VMEM((K,), jnp.int32),
        pltpu.VMEM((K,), jnp.int32),
        pltpu.VMEM((K,), jnp.int32),
        pltpu.VMEM((K,), jnp.int32),
        pltpu.VMEM((K, D), jnp.float32),
        pltpu.VMEM((K, D), jnp.float32),
        pltpu.VMEM_SHARED((AR, D), jnp.float32),
        pltpu.SemaphoreType.DMA,
        pltpu.SemaphoreType.DMA,
        pltpu.SemaphoreType.DMA,
        pltpu.SemaphoreType.DMA,
        pltpu.SemaphoreType.DMA,
        pltpu.SemaphoreType.DMA,
    ]
    if compute_cnt:
        out_type.append(jax.ShapeDtypeStruct((NC, AR), jnp.float32))
        scratch += [
            pltpu.VMEM((K,), jnp.float32),
            pltpu.VMEM((640,), jnp.float32),
            pltpu.VMEM_SHARED((AR,), jnp.float32),
        ]

    def body(x_hbm, ei_hbm, *refs):
        if compute_cnt:
            (agg_out, cnt_out, sidx_raw0, sidx_raw1, didx_raw0, didx_raw1,
             didx_adj0, didx_adj1, rows0, rows1, agg_sh,
             isem0, isem1, gsem0, gsem1, ssem0, ssem1,
             ones_v, zc_v, cnt_sh) = refs
        else:
            (agg_out, sidx_raw0, sidx_raw1, didx_raw0, didx_raw1,
             didx_adj0, didx_adj1, rows0, rows1, agg_sh,
             isem0, isem1, gsem0, gsem1, ssem0, ssem1) = refs
        c = lax.axis_index("c")
        s = lax.axis_index("s")
        w = s * NC + c
        sidx_raw = (sidx_raw0, sidx_raw1)
        didx_raw = (didx_raw0, didx_raw1)
        didx_adj = (didx_adj0, didx_adj1)
        rows = (rows0, rows1)
        isem = (isem0, isem1)
        gsem = (gsem0, gsem1)
        ssem = (ssem0, ssem1)

        def ebase(g):
            start = w * EPW + g * K
            return pl.multiple_of(jnp.minimum(start, E - K), 8)

        def fire_idx(g, b):
            pltpu.async_copy(ei_hbm.at[pl.ds(ebase(g), K)], sidx_raw[b],
                             isem[b])
            pltpu.async_copy(ei_hbm.at[pl.ds(E + ebase(g), K)],
                             didx_raw[b], isem[b])

        def wait_idx(g, b):
            pltpu.make_async_copy(ei_hbm.at[pl.ds(ebase(g), K)],
                                  sidx_raw[b], isem[b]).wait()
            pltpu.make_async_copy(ei_hbm.at[pl.ds(E + ebase(g), K)],
                                  didx_raw[b], isem[b]).wait()

        pad16 = N + lax.iota(jnp.int32, 16)
        iota16 = lax.iota(jnp.int32, 16)

        def adjust(g, b):
            start = w * EPW + g * K
            delta = start - jnp.minimum(start, E - K)

            def f(j, carry):
                d = didx_raw[b][pl.ds(j * 16, 16)]
                lane = j * 16 + iota16
                ok = (lane >= delta) & (lane + (g * K - delta) < EPW)
                didx_adj[b][pl.ds(j * 16, 16)] = jnp.where(ok, d, pad16)
                return carry

            lax.fori_loop(0, K // 16, f, 0)

        def fire_gather(b):
            pltpu.async_copy(x_hbm.at[sidx_raw[b]], rows[b], gsem[b])

        def wait_gather(b):
            pltpu.make_async_copy(x_hbm.at[sidx_raw[b]], rows[b],
                                  gsem[b]).wait()

        def fire_scatter(b):
            pltpu.async_copy(rows[b], agg_sh.at[didx_adj[b]], ssem[b],
                             add=True)
            if compute_cnt:
                pltpu.async_copy(ones_v, cnt_sh.at[didx_adj[b]], ssem[b],
                                 add=True)

        def wait_scatter(b):
            pltpu.make_async_copy(rows[b], agg_sh.at[didx_adj[b]],
                                  ssem[b]).wait()
            if compute_cnt:
                pltpu.make_async_copy(ones_v, cnt_sh.at[didx_adj[b]],
                                      ssem[b]).wait()

        fire_idx(0, 0)
        fire_idx(1, 1)

        zero16 = jnp.zeros((16,), jnp.float32)

        def zrow(r, carry):
            for j in range(D // 16):
                rows0[r, pl.ds(j * 16, 16)] = zero16
            return carry

        lax.fori_loop(0, K, zrow, 0)
        zrow0 = pl.multiple_of(s * ZR, 8 if ZR % 8 == 0 else 2)
        for t in range(3):
            pltpu.sync_copy(rows0, agg_sh.at[pl.ds(zrow0 + t * K, K)])
        pltpu.sync_copy(rows0.at[pl.ds(0, ZR - 3 * K)],
                        agg_sh.at[pl.ds(zrow0 + 3 * K, ZR - 3 * K)])

        if compute_cnt:
            one16 = jnp.ones((16,), jnp.float32)

            def fill_ones(r, carry):
                ones_v[pl.ds(r * 16, 16)] = one16
                return carry

            lax.fori_loop(0, K // 16, fill_ones, 0)

            def zc(r, carry):
                zc_v[pl.ds(r * 16, 16)] = zero16
                return carry

            lax.fori_loop(0, 640 // 16, zc, 0)
            pltpu.sync_copy(zc_v.at[pl.ds(0, ZR)],
                            cnt_sh.at[pl.ds(zrow0, ZR)])

        plsc.subcore_barrier()

        wait_idx(0, 0)
        adjust(0, 0)
        fire_gather(0)

        def step(i, carry):
            g0 = 2 * i
            g1 = 2 * i + 1
            wait_idx(g1, 1)

            @pl.when(i > 0)
            def _():
                wait_scatter(1)

            adjust(g1, 1)
            wait_gather(0)
            fire_gather(1)

            @pl.when(g0 + 2 < CHP)
            def _():
                fire_idx(g0 + 2, 0)

            fire_scatter(0)
            wait_gather(1)
            fire_scatter(1)
            wait_scatter(0)

            @pl.when(g0 + 2 < CHP)
            def _():
                wait_idx(g0 + 2, 0)
                adjust(g0 + 2, 0)
                fire_gather(0)

            @pl.when(g1 + 2 < CHP)
            def _():
                fire_idx(g1 + 2, 1)

            return carry

        lax.fori_loop(0, CHP // 2, step, 0)
        wait_scatter(1)
        plsc.subcore_barrier()

        crow0 = pl.multiple_of(s * CPT, 8)
        pltpu.sync_copy(agg_sh.at[pl.ds(crow0, CPT)],
                        agg_out.at[c, pl.ds(crow0, CPT)])

        @pl.when(s == 0)
        def _():
            pltpu.sync_copy(agg_sh.at[pl.ds(CPT * NS, CREM)],
                            agg_out.at[c, pl.ds(CPT * NS, CREM)])
            if compute_cnt:
                pltpu.sync_copy(cnt_sh, cnt_out.at[c])

    return pl.kernel(body, out_type=tuple(out_type), mesh=mesh,
                     scratch_types=tuple(scratch))


_sc_agg_cnt = _make_sc_aggregate(compute_cnt=True)
_sc_agg = _make_sc_aggregate(compute_cnt=False, K=192, CHP=54, AR=10016)

_TC_R = 2000


def _tc_root(x, wr, b):
    def body(x_ref, wr_ref, b_ref, o_ref):
        o_ref[...] = jax.lax.dot_general(
            x_ref[...], wr_ref[...], (((1,), (0,)), ((), ())),
            precision=jax.lax.Precision.HIGHEST,
            preferred_element_type=jnp.float32) + b_ref[...]

    return pl.pallas_call(
        body,
        grid=(N // _TC_R,),
        in_specs=[
            pl.BlockSpec((_TC_R, D), lambda i: (i, 0)),
            pl.BlockSpec((D, D), lambda i: (0, 0)),
            pl.BlockSpec((1, D), lambda i: (0, 0)),
        ],
        out_specs=pl.BlockSpec((_TC_R, D), lambda i: (i, 0)),
        out_shape=jax.ShapeDtypeStruct((N, D), jnp.float32),
    )(x, wr, b)


def _make_tc_combine(relu):
    def body(a0_ref, a1_ref, cnt_ref, xr_ref, wl_ref, o_ref):
        agg = (a0_ref[...] + a1_ref[...]) / jnp.maximum(cnt_ref[...], 1.0)
        acc = jax.lax.dot_general(
            agg, wl_ref[...], (((1,), (0,)), ((), ())),
            precision=jax.lax.Precision.HIGHEST,
            preferred_element_type=jnp.float32)
        acc = acc + xr_ref[...]
        if relu:
            acc = jnp.maximum(acc, 0.0)
        o_ref[...] = acc

    return pl.pallas_call(
        body,
        grid=(N // _TC_R,),
        in_specs=[
            pl.BlockSpec((_TC_R, D), lambda i: (i, 0)),
            pl.BlockSpec((_TC_R, D), lambda i: (i, 0)),
            pl.BlockSpec((_TC_R, 1), lambda i: (i, 0)),
            pl.BlockSpec((_TC_R, D), lambda i: (i, 0)),
            pl.BlockSpec((D, D), lambda i: (0, 0)),
        ],
        out_specs=pl.BlockSpec((_TC_R, D), lambda i: (i, 0)),
        out_shape=jax.ShapeDtypeStruct((N, D), jnp.float32),
    )


_tc_relu = _make_tc_combine(relu=True)
_tc_plain = _make_tc_combine(relu=False)


def kernel(x, edge_index, W1l, W1r, b1, W2l, W2r, b2):
    ei = edge_index.astype(jnp.int32).reshape(2 * E)
    agg1, cnt_p = _sc_agg_cnt(x, ei)
    xr1 = _tc_root(x, W1r, b1.reshape(1, D))
    cnt = (cnt_p[0, :N] + cnt_p[1, :N]).reshape(N, 1)
    h = _tc_relu(agg1[0], agg1[1], cnt, xr1, W1l)
    (agg2,) = _sc_agg(h, ei)
    xr2 = _tc_root(h, W2r, b2.reshape(1, D))
    out = _tc_plain(agg2[0], agg2[1], cnt, xr2, W2l)
    return out


# speedup vs baseline: 1.0062x; 1.0062x over previous
import jax
import jax.numpy as jnp
from jax import lax
from jax.experimental import pallas as pl
from jax.experimental.pallas import tpu as pltpu
from jax.experimental.pallas import tpu_sc as plsc

N = 10000
D = 128
E = 320000
NC = 2
NS = 16
NW = NC * NS
AR = 10112
EPW = E // NW
K = 176
CHP = 58
EPTP = CHP * K
ZR = AR // NS
CPT = 624
CREM = N - CPT * NS


def _make_sc_aggregate(compute_cnt, K=K, CHP=CHP, AR=AR):
    ZR = AR // NS
    mesh = plsc.VectorSubcoreMesh(core_axis_name="c", subcore_axis_name="s",
                                  num_cores=NC, num_subcores=NS)
    out_type = [jax.ShapeDtypeStruct((NC, N, D), jnp.float32)]
    scratch = [
        pltpu.VMEM((K,), jnp.int32),
        pltpu.VMEM((K,), jnp.int32),
        pltpu.VMEM((K,), jnp.int32),
        pltpu.VMEM((K,), jnp.int32),
        pltpu.VMEM((K,), jnp.int32),
        pltpu.VMEM((K,), jnp.int32),
        pltpu.VMEM((K, D), jnp.float32),
        pltpu.VMEM((K, D), jnp.float32),
        pltpu.VMEM_SHARED((AR, D), jnp.float32),
        pltpu.SemaphoreType.DMA,
        pltpu.SemaphoreType.DMA,
        pltpu.SemaphoreType.DMA,
        pltpu.SemaphoreType.DMA,
        pltpu.SemaphoreType.DMA,
        pltpu.SemaphoreType.DMA,
    ]
    if compute_cnt:
        out_type.append(jax.ShapeDtypeStruct((NC, AR), jnp.float32))
        scratch += [
            pltpu.VMEM((K,), jnp.float32),
            pltpu.VMEM((640,), jnp.float32),
            pltpu.VMEM_SHARED((AR,), jnp.float32),
        ]

    def body(x_hbm, ei_hbm, *refs):
        if compute_cnt:
            (agg_out, cnt_out, sidx_raw0, sidx_raw1, didx_raw0, didx_raw1,
             didx_adj0, didx_adj1, rows0, rows1, agg_sh,
             isem0, isem1, gsem0, gsem1, ssem0, ssem1,
             ones_v, zc_v, cnt_sh) = refs
        else:
            (agg_out, sidx_raw0, sidx_raw1, didx_raw0, didx_raw1,
             didx_adj0, didx_adj1, rows0, rows1, agg_sh,
             isem0, isem1, gsem0, gsem1, ssem0, ssem1) = refs
        c = lax.axis_index("c")
        s = lax.axis_index("s")
        w = s * NC + c
        sidx_raw = (sidx_raw0, sidx_raw1)
        didx_raw = (didx_raw0, didx_raw1)
        didx_adj = (didx_adj0, didx_adj1)
        rows = (rows0, rows1)
        isem = (isem0, isem1)
        gsem = (gsem0, gsem1)
        ssem = (ssem0, ssem1)

        def ebase(g):
            start = w * EPW + g * K
            return pl.multiple_of(jnp.minimum(start, E - K), 8)

        def fire_idx(g, b):
            pltpu.async_copy(ei_hbm.at[pl.ds(ebase(g), K)], sidx_raw[b],
                             isem[b])
            pltpu.async_copy(ei_hbm.at[pl.ds(E + ebase(g), K)],
                             didx_raw[b], isem[b])

        def wait_idx(g, b):
            pltpu.make_async_copy(ei_hbm.at[pl.ds(ebase(g), K)],
                                  sidx_raw[b], isem[b]).wait()
            pltpu.make_async_copy(ei_hbm.at[pl.ds(E + ebase(g), K)],
                                  didx_raw[b], isem[b]).wait()

        pad16 = N + lax.iota(jnp.int32, 16)
        iota16 = lax.iota(jnp.int32, 16)

        def adjust(g, b):
            start = w * EPW + g * K
            delta = start - jnp.minimum(start, E - K)

            def f(j, carry):
                d = didx_raw[b][pl.ds(j * 16, 16)]
                lane = j * 16 + iota16
                ok = (lane >= delta) & (lane + (g * K - delta) < EPW)
                didx_adj[b][pl.ds(j * 16, 16)] = jnp.where(ok, d, pad16)
                return carry

            lax.fori_loop(0, K // 16, f, 0)

        def fire_gather(b):
            pltpu.async_copy(x_hbm.at[sidx_raw[b]], rows[b], gsem[b])

        def wait_gather(b):
            pltpu.make_async_copy(x_hbm.at[sidx_raw[b]], rows[b],
                                  gsem[b]).wait()

        def fire_scatter(b):
            pltpu.async_copy(rows[b], agg_sh.at[didx_adj[b]], ssem[b],
                             add=True)
            if compute_cnt:
                pltpu.async_copy(ones_v, cnt_sh.at[didx_adj[b]], ssem[b],
                                 add=True)

        def wait_scatter(b):
            pltpu.make_async_copy(rows[b], agg_sh.at[didx_adj[b]],
                                  ssem[b]).wait()
            if compute_cnt:
                pltpu.make_async_copy(ones_v, cnt_sh.at[didx_adj[b]],
                                      ssem[b]).wait()

        fire_idx(0, 0)
        fire_idx(1, 1)

        zero16 = jnp.zeros((16,), jnp.float32)

        def zrow(r, carry):
            for j in range(D // 16):
                rows0[r, pl.ds(j * 16, 16)] = zero16
            return carry

        lax.fori_loop(0, K, zrow, 0)
        zrow0 = pl.multiple_of(s * ZR, 8 if ZR % 8 == 0 else 2)
        for t in range(3):
            pltpu.sync_copy(rows0, agg_sh.at[pl.ds(zrow0 + t * K, K)])
        pltpu.sync_copy(rows0.at[pl.ds(0, ZR - 3 * K)],
                        agg_sh.at[pl.ds(zrow0 + 3 * K, ZR - 3 * K)])

        if compute_cnt:
            one16 = jnp.ones((16,), jnp.float32)

            def fill_ones(r, carry):
                ones_v[pl.ds(r * 16, 16)] = one16
                return carry

            lax.fori_loop(0, K // 16, fill_ones, 0)

            def zc(r, carry):
                zc_v[pl.ds(r * 16, 16)] = zero16
                return carry

            lax.fori_loop(0, 640 // 16, zc, 0)
            pltpu.sync_copy(zc_v.at[pl.ds(0, ZR)],
                            cnt_sh.at[pl.ds(zrow0, ZR)])

        plsc.subcore_barrier()

        wait_idx(0, 0)
        adjust(0, 0)
        fire_gather(0)

        def step(i, carry):
            g0 = 2 * i
            g1 = 2 * i + 1
            wait_idx(g1, 1)

            @pl.when(i > 0)
            def _():
                wait_scatter(1)

            adjust(g1, 1)
            wait_gather(0)
            fire_gather(1)

            @pl.when(g0 + 2 < CHP)
            def _():
                fire_idx(g0 + 2, 0)

            fire_scatter(0)
            wait_gather(1)
            fire_scatter(1)
            wait_scatter(0)

            @pl.when(g0 + 2 < CHP)
            def _():
                wait_idx(g0 + 2, 0)
                adjust(g0 + 2, 0)
                fire_gather(0)

            @pl.when(g1 + 2 < CHP)
            def _():
                fire_idx(g1 + 2, 1)

            return carry

        lax.fori_loop(0, CHP // 2, step, 0)
        wait_scatter(1)
        plsc.subcore_barrier()

        crow0 = pl.multiple_of(s * CPT, 8)
        pltpu.sync_copy(agg_sh.at[pl.ds(crow0, CPT)],
                        agg_out.at[c, pl.ds(crow0, CPT)])

        @pl.when(s == 0)
        def _():
            pltpu.sync_copy(agg_sh.at[pl.ds(CPT * NS, CREM)],
                            agg_out.at[c, pl.ds(CPT * NS, CREM)])
            if compute_cnt:
                pltpu.sync_copy(cnt_sh, cnt_out.at[c])

    return pl.kernel(body, out_type=tuple(out_type), mesh=mesh,
                     scratch_types=tuple(scratch))


_sc_agg_cnt = _make_sc_aggregate(compute_cnt=True)
_sc_agg = _make_sc_aggregate(compute_cnt=False)

_TC_R = 2000


def _tc_root(x, wr, b):
    def body(x_ref, wr_ref, b_ref, o_ref):
        o_ref[...] = jax.lax.dot_general(
            x_ref[...], wr_ref[...], (((1,), (0,)), ((), ())),
            precision=jax.lax.Precision.HIGHEST,
            preferred_element_type=jnp.float32) + b_ref[...]

    return pl.pallas_call(
        body,
        grid=(N // _TC_R,),
        in_specs=[
            pl.BlockSpec((_TC_R, D), lambda i: (i, 0)),
            pl.BlockSpec((D, D), lambda i: (0, 0)),
            pl.BlockSpec((1, D), lambda i: (0, 0)),
        ],
        out_specs=pl.BlockSpec((_TC_R, D), lambda i: (i, 0)),
        out_shape=jax.ShapeDtypeStruct((N, D), jnp.float32),
    )(x, wr, b)


def _make_tc_combine(relu):
    def body(a0_ref, a1_ref, cnt_ref, xr_ref, wl_ref, o_ref):
        agg = (a0_ref[...] + a1_ref[...]) / jnp.maximum(cnt_ref[...], 1.0)
        acc = jax.lax.dot_general(
            agg, wl_ref[...], (((1,), (0,)), ((), ())),
            precision=jax.lax.Precision.HIGHEST,
            preferred_element_type=jnp.float32)
        acc = acc + xr_ref[...]
        if relu:
            acc = jnp.maximum(acc, 0.0)
        o_ref[...] = acc

    return pl.pallas_call(
        body,
        grid=(N // _TC_R,),
        in_specs=[
            pl.BlockSpec((_TC_R, D), lambda i: (i, 0)),
            pl.BlockSpec((_TC_R, D), lambda i: (i, 0)),
            pl.BlockSpec((_TC_R, 1), lambda i: (i, 0)),
            pl.BlockSpec((_TC_R, D), lambda i: (i, 0)),
            pl.BlockSpec((D, D), lambda i: (0, 0)),
        ],
        out_specs=pl.BlockSpec((_TC_R, D), lambda i: (i, 0)),
        out_shape=jax.ShapeDtypeStruct((N, D), jnp.float32),
    )


_tc_relu = _make_tc_combine(relu=True)
_tc_plain = _make_tc_combine(relu=False)


def kernel(x, edge_index, W1l, W1r, b1, W2l, W2r, b2):
    ei = edge_index.astype(jnp.int32).reshape(2 * E)
    agg1, cnt_p = _sc_agg_cnt(x, ei)
    xr1 = _tc_root(x, W1r, b1.reshape(1, D))
    cnt = (cnt_p[0, :N] + cnt_p[1, :N]).reshape(N, 1)
    h = _tc_relu(agg1[0], agg1[1], cnt, xr1, W1l)
    (agg2,) = _sc_agg(h, ei)
    xr2 = _tc_root(h, W2r, b2.reshape(1, D))
    out = _tc_plain(agg2[0], agg2[1], cnt, xr2, W2l)
    return out
